# baseline (speedup 1.0000x reference)
import jax
import jax.numpy as jnp
from jax.experimental import pallas as pl
from jax.experimental.pallas import tpu as pltpu

_BLOCK_R = 8192
_SUB_R = 1024


def _body(ptr_ref, lo_ref, hi_ref, x_ref, w_ref, b_ref, o_ref):
    i = pl.program_id(0)
    r = x_ref.shape[0]
    nseg = o_ref.shape[0]
    first = ptr_ref[0] // r
    last = jnp.maximum(ptr_ref[nseg] - 1, ptr_ref[0]) // r
    j = first + i

    @pl.when(i == 0)
    def _init():
        o_ref[...] = jnp.zeros_like(o_ref)

    @pl.when(j <= last)
    def _acc():
        wb = w_ref[...].astype(jnp.bfloat16)
        bias = b_ref[...]
        lo = lo_ref[...]
        hi = hi_ref[...]
        t = _SUB_R

        def sub(k, acc):
            xs = x_ref[pl.ds(k * t, t), :].astype(jnp.bfloat16)
            h = jax.lax.dot_general(
                xs, wb, (((1,), (1,)), ((), ())),
                preferred_element_type=jnp.float32)
            h = h + bias
            h = jnp.maximum(h, 0.2 * h)
            pos = (j * r + k * t
                   + jax.lax.broadcasted_iota(jnp.int32, (nseg, t), 1))
            sel = ((pos >= lo) & (pos < hi)).astype(jnp.float32)
            return acc + jnp.dot(sel, h, preferred_element_type=jnp.float32)

        o_ref[...] += jax.lax.fori_loop(
            0, r // t, sub, jnp.zeros_like(o_ref))


def kernel(h_dag, obs_ptr, W, b):
    n, d = h_dag.shape
    nseg = obs_ptr.shape[0] - 1
    r = _BLOCK_R
    lo = obs_ptr[:-1].reshape(nseg, 1)
    hi = obs_ptr[1:].reshape(nseg, 1)

    def x_map(i, ptr):
        first = ptr[0] // r
        last = jnp.maximum(ptr[nseg] - 1, ptr[0]) // r
        return (jnp.minimum(first + i, last), 0)

    grid_spec = pltpu.PrefetchScalarGridSpec(
        num_scalar_prefetch=1,
        grid=(n // r,),
        in_specs=[
            pl.BlockSpec((nseg, 1), lambda i, ptr: (0, 0)),
            pl.BlockSpec((nseg, 1), lambda i, ptr: (0, 0)),
            pl.BlockSpec((r, d), x_map),
            pl.BlockSpec((d, d), lambda i, ptr: (0, 0)),
            pl.BlockSpec((1, d), lambda i, ptr: (0, 0)),
        ],
        out_specs=pl.BlockSpec((nseg, d), lambda i, ptr: (0, 0)),
    )
    return pl.pallas_call(
        _body,
        grid_spec=grid_spec,
        out_shape=jax.ShapeDtypeStruct((nseg, d), jnp.float32),
    )(obs_ptr, lo, hi, h_dag, W, b.reshape(1, d))


# speedup vs baseline: 1.4214x; 1.4214x over previous
import jax
import jax.numpy as jnp
from jax.experimental import pallas as pl
from jax.experimental.pallas import tpu as pltpu

_HALF_R = 4096


def _body(ptr_ref, lo_ref, hi_ref, xa_ref, xb_ref, w_ref, b_ref, o_ref):
    i = pl.program_id(0)
    rh = xa_ref.shape[0]
    rs = 2 * rh
    nseg = o_ref.shape[0]
    first = ptr_ref[0] // rs
    last = jnp.maximum(ptr_ref[nseg] - 1, ptr_ref[0]) // rs
    j = first + i

    @pl.when(i == 0)
    def _init():
        o_ref[...] = jnp.zeros_like(o_ref)

    @pl.when(j <= last)
    def _acc():
        wb = w_ref[...].astype(jnp.bfloat16)
        bias = b_ref[...]
        lo = lo_ref[...]
        hi = hi_ref[...]

        def half(x_ref, base):
            h = jax.lax.dot_general(
                x_ref[...].astype(jnp.bfloat16), wb,
                (((1,), (1,)), ((), ())),
                preferred_element_type=jnp.float32)
            h = h + bias
            h = jnp.maximum(h, 0.2 * h)
            pos = base + jax.lax.broadcasted_iota(jnp.int32, (nseg, rh), 1)
            sel = ((pos >= lo) & (pos < hi)).astype(jnp.float32)
            return jnp.dot(sel, h, preferred_element_type=jnp.float32)

        o_ref[...] += half(xa_ref, j * rs) + half(xb_ref, j * rs + rh)


def kernel(h_dag, obs_ptr, W, b):
    n, d = h_dag.shape
    nseg = obs_ptr.shape[0] - 1
    rh = _HALF_R
    rs = 2 * rh
    lo = obs_ptr[:-1].reshape(nseg, 1)
    hi = obs_ptr[1:].reshape(nseg, 1)

    def _super(i, ptr):
        first = ptr[0] // rs
        last = jnp.maximum(ptr[nseg] - 1, ptr[0]) // rs
        return jnp.minimum(first + i, last)

    grid_spec = pltpu.PrefetchScalarGridSpec(
        num_scalar_prefetch=1,
        grid=(n // rs,),
        in_specs=[
            pl.BlockSpec((nseg, 1), lambda i, ptr: (0, 0)),
            pl.BlockSpec((nseg, 1), lambda i, ptr: (0, 0)),
            pl.BlockSpec((rh, d), lambda i, ptr: (2 * _super(i, ptr), 0)),
            pl.BlockSpec((rh, d), lambda i, ptr: (2 * _super(i, ptr) + 1, 0)),
            pl.BlockSpec((d, d), lambda i, ptr: (0, 0)),
            pl.BlockSpec((1, d), lambda i, ptr: (0, 0)),
        ],
        out_specs=pl.BlockSpec((nseg, d), lambda i, ptr: (0, 0)),
    )
    return pl.pallas_call(
        _body,
        grid_spec=grid_spec,
        out_shape=jax.ShapeDtypeStruct((nseg, d), jnp.float32),
    )(obs_ptr, lo, hi, h_dag, h_dag, W, b.reshape(1, d))


# speedup vs baseline: 1.4418x; 1.0143x over previous
import jax
import jax.numpy as jnp
from jax.experimental import pallas as pl
from jax.experimental.pallas import tpu as pltpu

_BLOCK_R = 8192


def _body(ptr_ref, lo_ref, hi_ref, x_ref, w_ref, b_ref, o_ref):
    i = pl.program_id(0)
    r = x_ref.shape[0]
    nseg = o_ref.shape[1]
    first = ptr_ref[0] // r
    last = jnp.maximum(ptr_ref[nseg] - 1, ptr_ref[0]) // r
    j = first + i

    @pl.when(j <= last)
    def _acc():
        h = jax.lax.dot_general(
            x_ref[...].astype(jnp.bfloat16), w_ref[...].astype(jnp.bfloat16),
            (((1,), (1,)), ((), ())),
            preferred_element_type=jnp.float32)
        h = h + b_ref[...]
        h = jnp.maximum(h, 0.2 * h)
        pos = j * r + jax.lax.broadcasted_iota(jnp.int32, (nseg, r), 1)
        sel = ((pos >= lo_ref[...]) & (pos < hi_ref[...])).astype(jnp.float32)
        o_ref[0] = jnp.dot(sel, h, preferred_element_type=jnp.float32)

    @pl.when(j > last)
    def _zero():
        o_ref[...] = jnp.zeros_like(o_ref)


def kernel(h_dag, obs_ptr, W, b):
    n, d = h_dag.shape
    nseg = obs_ptr.shape[0] - 1
    r = _BLOCK_R
    lo = obs_ptr[:-1].reshape(nseg, 1)
    hi = obs_ptr[1:].reshape(nseg, 1)

    def x_map(i, ptr):
        first = ptr[0] // r
        last = jnp.maximum(ptr[nseg] - 1, ptr[0]) // r
        return (jnp.minimum(first + i, last), 0)

    grid_spec = pltpu.PrefetchScalarGridSpec(
        num_scalar_prefetch=1,
        grid=(n // r,),
        in_specs=[
            pl.BlockSpec((nseg, 1), lambda i, ptr: (0, 0)),
            pl.BlockSpec((nseg, 1), lambda i, ptr: (0, 0)),
            pl.BlockSpec((r, d), x_map),
            pl.BlockSpec((d, d), lambda i, ptr: (0, 0)),
            pl.BlockSpec((1, d), lambda i, ptr: (0, 0)),
        ],
        out_specs=pl.BlockSpec((1, nseg, d), lambda i, ptr: (i, 0, 0)),
    )
    partials = pl.pallas_call(
        _body,
        grid_spec=grid_spec,
        out_shape=jax.ShapeDtypeStruct((n // r, nseg, d), jnp.float32),
    )(obs_ptr, lo, hi, h_dag, W, b.reshape(1, d))
    return partials.sum(axis=0)
